# baseline (speedup 1.0000x reference)
import functools

import jax
import jax.numpy as jnp
from jax import lax
from jax.experimental import pallas as pl
from jax.experimental.pallas import tpu as pltpu
from jax.experimental.pallas import tpu_sc as plsc

_EMBED_DIM = 32
_BATCH = 16384

_NC = 2
_NS = 16
_NW = _NC * _NS
_BPW = _BATCH // _NW
_CHUNK = 128
_NCHUNK = _BPW // _CHUNK
_PACK = 128 // _EMBED_DIM


@jax.jit
def _gather(table128, idx):
  mesh = plsc.VectorSubcoreMesh(core_axis_name="c", subcore_axis_name="s")

  @functools.partial(
      pl.kernel,
      mesh=mesh,
      out_type=jax.ShapeDtypeStruct((_BATCH // _PACK, 128), jnp.float32),
      scratch_types=[
          pltpu.VMEM((_BPW + 16,), jnp.int32),
          pltpu.VMEM((_NCHUNK, _CHUNK), jnp.int32),
          pltpu.VMEM((_BPW, 128), jnp.float32),
          pltpu.VMEM((_BPW // _PACK, 128), jnp.float32),
          pltpu.SemaphoreType.DMA,
      ],
  )
  def k(table_hbm, idx_hbm, out_hbm, idx_v, phys_v, rows_v, out_v, sem):
    wid = lax.axis_index("s") * _NC + lax.axis_index("c")
    pltpu.sync_copy(idx_hbm.at[wid], idx_v.at[pl.ds(0, _BPW)])
    two = jnp.int32(2)
    for j in range(_NCHUNK):
      for t in range(_CHUNK // 16):
        phys_v[j, pl.ds(t * 16, 16)] = lax.shift_right_logical(
            idx_v[pl.ds(j * _CHUNK + t * 16, 16)], two)
    copies = []
    for j in range(_NCHUNK):
      copies.append(
          pltpu.async_copy(
              table_hbm.at[phys_v.at[j]],
              rows_v.at[pl.ds(j * _CHUNK, _CHUNK)],
              sem,
          ))
    for c in copies:
      c.wait()

    def body(q, _):
      rv = idx_v[pl.ds(q * _PACK, 16)]
      subv = (rv & 3) * 32
      for t in range(_PACK):
        i = q * _PACK + t
        sub = subv[t]
        out_v[q, pl.ds(t * 32, 16)] = rows_v[i, pl.ds(sub, 16)]
        out_v[q, pl.ds(t * 32 + 16, 16)] = rows_v[i, pl.ds(sub + 16, 16)]
      return 0

    lax.fori_loop(0, _BPW // _PACK, body, 0)
    pltpu.sync_copy(out_v, out_hbm.at[pl.ds(wid * (_BPW // _PACK),
                                            _BPW // _PACK)])

  return k(table128, idx)


def kernel(item_fea, table):
  idx = item_fea[:, 0].astype(jnp.int32).reshape(_NW, _BPW)
  table128 = table.reshape(table.shape[0] // _PACK, 128)
  out = _gather(table128, idx)
  return out.reshape(_BATCH, _EMBED_DIM)
